# baseline (speedup 1.0000x reference)
import functools

import jax
import jax.numpy as jnp
from jax import lax
from jax.experimental import pallas as pl
from jax.experimental.pallas import tpu as pltpu
from jax.experimental.pallas import tpu_sc as plsc

NUM_CORES = 2
NUM_SUBCORES = 16
NW = NUM_CORES * NUM_SUBCORES
LANE = 128
BL = 128
PACK = 4


def _build_repack(vocab: int, d_model: int):
    mesh = plsc.VectorSubcoreMesh(
        core_axis_name="c", subcore_axis_name="s",
        num_cores=NUM_CORES, num_subcores=NUM_SUBCORES)
    packed_rows = vocab // PACK
    full_blocks = packed_rows // BL
    tail_rows = packed_rows - full_blocks * BL
    per_w = full_blocks // NW
    rem_blocks = full_blocks - per_w * NW

    @functools.partial(
        pl.kernel,
        out_type=jax.ShapeDtypeStruct((packed_rows, LANE), jnp.float32),
        mesh=mesh,
        scratch_types=[
            pltpu.VMEM((2, d_model, PACK * BL), jnp.float32),
            pltpu.VMEM((2, BL, LANE), jnp.float32),
            pltpu.SemaphoreType.DMA,
            pltpu.SemaphoreType.DMA,
            pltpu.SemaphoreType.DMA,
            pltpu.SemaphoreType.DMA,
        ],
        compiler_params=pltpu.CompilerParams(
            needs_layout_passes=False, disable_bounds_checks=True),
    )
    def repack_kernel(tt_hbm, tail_hbm, tp_hbm, nbuf, pbuf, g0, g1, s0, s1):
        wid = lax.axis_index("s") * NUM_CORES + lax.axis_index("c")
        gsems = (g0, g1)
        ssems = (s0, s1)
        iota = lax.iota(jnp.int32, 16)
        zeros = jnp.zeros((16,), jnp.int32)
        dvecs = [zeros + (16 * (k & 1)) + iota for k in range(8)]

        def read_desc(blk, b, ncols):
            return pltpu.make_async_copy(
                tt_hbm.at[:, pl.ds(blk * (PACK * BL), ncols)],
                nbuf.at[b].at[:, pl.ds(0, ncols)], gsems[b])

        def write_desc(blk, b, nrows):
            return pltpu.make_async_copy(
                pbuf.at[b].at[pl.ds(0, nrows)],
                tp_hbm.at[pl.ds(blk * BL, nrows)], ssems[b])

        def transpose(b, nrows, col_off=0):
            src = nbuf.at[b]
            dst = pbuf.at[b]

            @pl.loop(0, nrows)
            def _(p):
                base = col_off + p * PACK
                for k in range(8):
                    cols = zeros + (base + (k >> 1))
                    dst[p, pl.ds(16 * k, 16)] = plsc.load_gather(
                        src, [dvecs[k], cols])

        def do_block(blk, b, first, last):
            read_desc(blk, b, PACK * BL).wait()

            @pl.when(jnp.logical_not(first))
            def _():
                write_desc(blk, b, BL).wait()
            transpose(b, BL)
            write_desc(blk, b, BL).start()

            @pl.when(jnp.logical_not(last))
            def _():
                read_desc(blk + 2, b, PACK * BL).start()

        lo = wid * per_w
        read_desc(lo, 0, PACK * BL).start()
        read_desc(lo + 1, 1, PACK * BL).start()

        @pl.loop(0, per_w)
        def _(j):
            blk = lo + j
            b = lax.rem(j, 2)

            @pl.when(b == 0)
            def _():
                do_block(blk, 0, j == 0, j + 2 >= per_w)

            @pl.when(b == 1)
            def _():
                do_block(blk, 1, j == 1, j + 2 >= per_w)

        write_desc(0, 0, BL).wait()
        write_desc(0, 1, BL).wait()

        @pl.when(wid == 0)
        def _():
            blk = NW * per_w
            read_desc(blk, 0, PACK * BL).start()
            read_desc(blk, 0, PACK * BL).wait()
            transpose(0, BL)
            write_desc(blk, 0, BL).start()
            write_desc(blk, 0, BL).wait()

        @pl.when(wid == 1)
        def _():
            pltpu.sync_copy(
                tail_hbm, tp_hbm.at[pl.ds(full_blocks * BL, tail_rows)])

    return repack_kernel


def _build_gather(batch: int, seq: int, d_model: int):
    mesh = plsc.VectorSubcoreMesh(
        core_axis_name="c", subcore_axis_name="s",
        num_cores=NUM_CORES, num_subcores=NUM_SUBCORES)

    @functools.partial(
        pl.kernel,
        out_type=jax.ShapeDtypeStruct((seq, d_model, batch), jnp.float32),
        mesh=mesh,
        scratch_types=[
            pltpu.VMEM((seq, BL), jnp.int32),
            pltpu.VMEM((seq, BL), jnp.int32),
            pltpu.VMEM((4, BL, LANE), jnp.float32),
            pltpu.VMEM((4, d_model, BL), jnp.float32),
            pltpu.SemaphoreType.DMA,
            pltpu.SemaphoreType.DMA,
            pltpu.SemaphoreType.DMA,
            pltpu.SemaphoreType.DMA,
            pltpu.SemaphoreType.DMA,
            pltpu.SemaphoreType.DMA,
            pltpu.SemaphoreType.DMA,
            pltpu.SemaphoreType.DMA,
        ],
        compiler_params=pltpu.CompilerParams(needs_layout_passes=False),
    )
    def gather_kernel(xt_hbm, tp_hbm, out_hbm, xv, xq, rbuf, tbuf, *sems):
        wid = lax.axis_index("s") * NUM_CORES + lax.axis_index("c")
        b0 = wid * BL
        pltpu.sync_copy(xt_hbm.at[:, pl.ds(b0, BL)], xv)
        gsems = sems[:4]
        ssems = sems[4:]
        iota = lax.iota(jnp.int32, 16)
        nring = 4
        nstep = seq // nring
        ntail = seq - nstep * nring

        @pl.loop(0, seq)
        def _(s):
            for lb in range(BL // 16):
                v = xv[s, pl.ds(16 * lb, 16)]
                xq[s, pl.ds(16 * lb, 16)] = lax.shift_right_logical(v, 2)
                xv[s, pl.ds(16 * lb, 16)] = lax.shift_left(
                    jnp.bitwise_and(v, 3), 5)

        def gather_desc(s, b):
            return pltpu.make_async_copy(
                tp_hbm.at[xq.at[s]], rbuf.at[b], gsems[b])

        def store_desc(s, b):
            return pltpu.make_async_copy(
                tbuf.at[b], out_hbm.at[s].at[:, pl.ds(b0, BL)], ssems[b])

        def transpose(s, b):
            src = rbuf.at[b]
            dst = tbuf.at[b]
            for lb in range(BL // 16):
                rows = iota + (16 * lb)
                qcol = xv[s, pl.ds(16 * lb, 16)]
                for d in range(d_model):
                    dst[d, pl.ds(16 * lb, 16)] = plsc.load_gather(
                        src, [rows, qcol + d])

        for b in range(nring):
            gather_desc(b, b).start()

        @pl.loop(0, nstep)
        def _(h):
            h0 = h * nring
            for b in range(nring):
                s = h0 + b
                gather_desc(s, b).wait()

                @pl.when(h > 0)
                def _():
                    store_desc(s, b).wait()
                transpose(s, b)
                store_desc(s, b).start()

                @pl.when(s + nring < seq)
                def _():
                    gather_desc(s + nring, b).start()

        for b in range(nring):
            s_prev = nstep * nring - nring + b
            if b < ntail:
                st = nstep * nring + b
                store_desc(st, b).wait()
                gather_desc(st, b).wait()
                transpose(st, b)
                store_desc(st, b).start()
                store_desc(st, b).wait()
            else:
                store_desc(s_prev, b).wait()

    return gather_kernel


def kernel(x, embedding_matrix):
    batch, seq = x.shape
    vocab, d_model = embedding_matrix.shape
    repack = _build_repack(vocab, d_model)
    n_tail_rows = vocab - (vocab // (PACK * BL)) * (PACK * BL)
    tail = embedding_matrix[vocab - n_tail_rows:].reshape(
        n_tail_rows // PACK, LANE)
    tp = repack(embedding_matrix.T, tail)
    gather = _build_gather(batch, seq, d_model)
    out_t = gather(x.T, tp)
    return out_t.transpose(2, 0, 1)


# speedup vs baseline: 1.3013x; 1.3013x over previous
import functools

import jax
import jax.numpy as jnp
from jax import lax
from jax.experimental import pallas as pl
from jax.experimental.pallas import tpu as pltpu
from jax.experimental.pallas import tpu_sc as plsc

NUM_CORES = 2
NUM_SUBCORES = 16
NW = NUM_CORES * NUM_SUBCORES
GRP = 8


def _vec_blocks(seq):
    blocks = []
    o = 0
    while o + 16 <= seq:
        blocks.append(o)
        o += 16
    if o < seq:
        blocks.append(seq - 16)
    return blocks


def _build_gather(batch: int, seq: int, d_model: int):
    mesh = plsc.VectorSubcoreMesh(
        core_axis_name="c", subcore_axis_name="s",
        num_cores=NUM_CORES, num_subcores=NUM_SUBCORES)
    rows_per_w = batch // NW
    blocks = _vec_blocks(seq)

    @functools.partial(
        pl.kernel,
        out_type=jax.ShapeDtypeStruct((batch, seq, d_model), jnp.float32),
        mesh=mesh,
        scratch_types=[
            pltpu.VMEM((rows_per_w, seq), jnp.int32),
            pltpu.VMEM((seq - seq // 2, GRP, d_model), jnp.float32),
            pltpu.VMEM((seq - seq // 2, GRP, d_model), jnp.float32),
            pltpu.VMEM((seq, d_model), jnp.float32),
            pltpu.VMEM((seq, d_model), jnp.float32),
            pltpu.SemaphoreType.DMA,
            pltpu.SemaphoreType.DMA,
            pltpu.SemaphoreType.DMA,
            pltpu.SemaphoreType.DMA,
        ],
        compiler_params=pltpu.CompilerParams(needs_layout_passes=False),
    )
    def gather_kernel(x_hbm, t_hbm, out_hbm, xv, rbuf0, rbuf1,
                      obuf0, obuf1, g0, g1, s0, s1):
        wid = lax.axis_index("s") * NUM_CORES + lax.axis_index("c")
        r0 = wid * rows_per_w
        pltpu.sync_copy(x_hbm.at[pl.ds(r0, rows_per_w)], xv)
        rbufs = (rbuf0, rbuf1)
        obufs = (obuf0, obuf1)
        gsems = (g0, g1)
        ssems = (s0, s1)

        half_lo = (0, seq // 2)
        half_hi = (seq // 2, seq)

        def each_index(b, half, fn):
            lo, hi = half_lo[half], half_hi[half]
            done = set()
            o = lo
            offs = []
            while o + 16 <= hi:
                offs.append(o)
                o += 16
            if o < hi:
                offs.append(hi - 16)
            for o in offs:
                v = xv[b, pl.ds(o, 16)]
                for j in range(16):
                    s = o + j
                    if s < lo or s in done:
                        continue
                    done.add(s)
                    fn(s, v[j])

        def issue_gathers(b, half, buf):
            lo = half_lo[half]

            def fire(s, i):
                j8 = pl.multiple_of(
                    lax.shift_left(lax.shift_right_logical(i, 3), 3), GRP)
                pltpu.make_async_copy(
                    t_hbm.at[pl.ds(j8, GRP)], rbufs[buf].at[s - lo],
                    gsems[buf]).start()
            each_index(b, half, fire)

        def drain_gathers(half, buf):
            n = half_hi[half] - half_lo[half]
            for k in range(n):
                pltpu.make_async_copy(
                    t_hbm.at[pl.ds(0, GRP)], rbufs[buf].at[k],
                    gsems[buf]).wait()

        def select(b, half, buf, ob):
            lo = half_lo[half]

            def pick(s, i):
                q = jnp.bitwise_and(i, GRP - 1)
                for o in range(0, d_model, 16):
                    obufs[ob][s, pl.ds(o, 16)] = (
                        rbufs[buf][s - lo, q, pl.ds(o, 16)])
            each_index(b, half, pick)

        def store_desc(b, ob):
            return pltpu.make_async_copy(
                obufs[ob], out_hbm.at[r0 + b], ssems[ob])

        issue_gathers(0, 0, 0)
        issue_gathers(0, 1, 1)

        @pl.loop(0, rows_per_w // 2)
        def _(h):
            b = h * 2
            for ob in range(2):
                row = b + ob

                @pl.when(h > 0)
                def _():
                    store_desc(row, ob).wait()
                for half in range(2):
                    drain_gathers(half, half)
                    select(row, half, half, ob)

                    @pl.when(row + 1 < rows_per_w)
                    def _():
                        issue_gathers(row + 1, half, half)
                store_desc(row, ob).start()

            @pl.when(b + 2 >= rows_per_w)
            def _():
                store_desc(b, 0).wait()
                store_desc(b + 1, 1).wait()

    return gather_kernel


def kernel(x, embedding_matrix):
    batch, seq = x.shape
    _, d_model = embedding_matrix.shape
    gather = _build_gather(batch, seq, d_model)
    return gather(x, embedding_matrix)


# speedup vs baseline: 1.5096x; 1.1601x over previous
import functools

import jax
import jax.numpy as jnp
from jax import lax
from jax.experimental import pallas as pl
from jax.experimental.pallas import tpu as pltpu
from jax.experimental.pallas import tpu_sc as plsc

NUM_CORES = 2
NUM_SUBCORES = 16
NW = NUM_CORES * NUM_SUBCORES

NBUF = 8


def _build_gather(batch: int, seq: int, d_model: int):
    mesh = plsc.VectorSubcoreMesh(
        core_axis_name="c", subcore_axis_name="s",
        num_cores=NUM_CORES, num_subcores=NUM_SUBCORES)
    rows_per_w = batch // NW
    n_chunks = rows_per_w
    n_groups = n_chunks // NBUF

    @functools.partial(
        pl.kernel,
        out_type=jax.ShapeDtypeStruct((batch, seq, d_model), jnp.float32),
        mesh=mesh,
        scratch_types=[
            pltpu.VMEM((n_chunks, seq), jnp.int32),
            pltpu.VMEM((NBUF, seq, d_model), jnp.float32),
            pltpu.SemaphoreType.DMA,
            pltpu.SemaphoreType.DMA,
        ],
        compiler_params=pltpu.CompilerParams(use_tc_tiling_on_sc=False),
    )
    def gather_kernel(x_hbm, table_hbm, out_hbm, idx_v, rows_v, gsem, ssem):
        wid = lax.axis_index("s") * NUM_CORES + lax.axis_index("c")
        r0 = wid * rows_per_w
        pltpu.sync_copy(x_hbm.at[pl.ds(r0, rows_per_w)], idx_v)

        def gather_desc(j, b):
            return pltpu.make_async_copy(
                table_hbm.at[idx_v.at[j]], rows_v.at[b], gsem)

        def store_desc(j, b):
            return pltpu.make_async_copy(
                rows_v.at[b], out_hbm.at[r0 + j], ssem)

        for b in range(NBUF):
            gather_desc(b, b).start()

        @pl.loop(0, n_groups)
        def _(g):
            j0 = g * NBUF
            for b in range(NBUF):
                gather_desc(j0 + b, b).wait()
                store_desc(j0 + b, b).start()
            @pl.when(g + 1 < n_groups)
            def _():
                for b in range(NBUF):
                    store_desc(j0 + b, b).wait()
                    gather_desc(j0 + NBUF + b, b).start()

            @pl.when(g + 1 == n_groups)
            def _():
                for b in range(NBUF):
                    store_desc(j0 + b, b).wait()

    return gather_kernel


def kernel(x, embedding_matrix):
    batch, seq = x.shape
    _, d_model = embedding_matrix.shape
    gather = _build_gather(batch, seq, d_model)
    return gather(x, embedding_matrix)
